# baseline (speedup 1.0000x reference)
import functools

import jax
import jax.numpy as jnp
from jax import lax
from jax.experimental import pallas as pl
from jax.experimental.pallas import tpu as pltpu
from jax.experimental.pallas import tpu_sc as plsc

_LOG1P_C = (
    3.7050701212137938e-03,
    -2.2747693583369255e-02,
    6.5802522003650665e-02,
    -1.2435103952884674e-01,
    1.8400530517101288e-01,
    -2.4605530500411987e-01,
    3.3274200558662415e-01,
    -4.9995198845863342e-01,
    9.9999833106994629e-01,
    1.4770298761845880e-08,
)


def _log1p01(u):
    p = jnp.full((16,), _LOG1P_C[0], jnp.float32)
    for c in _LOG1P_C[1:]:
        p = p * u + c
    return p


def _sp100(x):
    l = _log1p01(jnp.exp(-jnp.abs(x)))
    return jnp.minimum(jnp.maximum(x, 0.0) + l, 100.0)


def _sp_both(x):
    l = _log1p01(jnp.exp(-jnp.abs(x)))
    sp_p = jnp.minimum(jnp.maximum(x, 0.0) + l, 100.0)
    sp_n = jnp.minimum(jnp.maximum(-x, 0.0) + l, 100.0)
    return sp_p, sp_n


def _rot(v, lane, s):
    return v.at[(lane + s) & 15].get(mode="promise_in_bounds")


def _lanesum(v):
    lane = lax.iota(jnp.int32, 16)
    for s in (1, 2, 4, 8):
        v = v + v.at[lane ^ s].get(mode="promise_in_bounds")
    return v


def _conf_dense_tc(pt):
    C_, B_, HW_ = pt.shape

    def body(ref, o_ref):
        x = ref[0]
        l = jnp.log1p(jnp.exp(-jnp.abs(x)))
        sp = jnp.minimum(jnp.maximum(x, 0.0) + l, 100.0)
        o_ref[0, 0] = jnp.sum(sp)

    return pl.pallas_call(
        body,
        grid=(1,),
        in_specs=[pl.BlockSpec((1, B_, HW_), lambda i: (4, 0, 0))],
        out_specs=pl.BlockSpec((1, 1), lambda i: (0, 0), memory_space=pltpu.SMEM),
        out_shape=jax.ShapeDtypeStruct((1, 1), jnp.float32),
    )(pt)[0, 0]


def _build_sc_call(B, HW, C, T):
    NCLS = C - 5
    NSLOT = C + 1
    NENT = NSLOT * 16
    NTCH = -(-NENT // 128)
    info = plsc.get_sparse_core_info()
    NC, NS = info.num_cores, info.num_subcores
    NW = NC * NS
    CELLS = B * HW
    CPT = CELLS // NW
    TP = 32
    mesh = plsc.VectorSubcoreMesh(core_axis_name="c", subcore_axis_name="s")

    @functools.partial(
        pl.kernel,
        mesh=mesh,
        out_type=jax.ShapeDtypeStruct((NW, 16), jnp.float32),
        scratch_types=[
            pltpu.VMEM((NTCH * 128,), jnp.int32),
            pltpu.VMEM((NTCH * 128,), jnp.float32),
            pltpu.VMEM((5 * TP,), jnp.float32),
            pltpu.VMEM((32,), jnp.float32),
            pltpu.VMEM((16,), jnp.float32),
            pltpu.SemaphoreType.DMA,
            pltpu.SemaphoreType.DMA,
        ],
    )
    def sc_fn(flat_hbm, tgt_hbm, grid_hbm, out_hbm,
              tgidx, tgbuf, tgt_v, grid_v, res_v, sem_c, sem_r):
        wid = lax.axis_index("s") * NC + lax.axis_index("c")
        batch = wid // 2
        half = wid % 2
        lane = lax.iota(jnp.int32, 16)
        zero16 = jnp.zeros((16,), jnp.float32)
        c16 = lambda k: jnp.full((16,), k, jnp.int32)

        pltpu.sync_copy(tgt_hbm.at[batch], tgt_v)
        pltpu.sync_copy(grid_hbm, grid_v)
        wf = grid_v[pl.ds(0, 16)]
        hf = grid_v[pl.ds(16, 16)]
        wi = wf.astype(jnp.int32)

        halves = []
        for h2 in range(2):
            clsf = tgt_v[pl.ds(0 * TP + 16 * h2, 16)]
            cx = tgt_v[pl.ds(1 * TP + 16 * h2, 16)]
            cy = tgt_v[pl.ds(2 * TP + 16 * h2, 16)]
            tw = tgt_v[pl.ds(3 * TP + 16 * h2, 16)]
            th = tgt_v[pl.ds(4 * TP + 16 * h2, 16)]
            gx = (cx * wf).astype(jnp.int32)
            gy = (cy * hf).astype(jnp.int32)
            cell = gy * wi + gx + batch * HW
            valid = (lane + 16 * h2) < T
            halves.append((clsf, cx, cy, tw, th, gx, gy, cell, valid))

        own_is0 = half == 0

        def sel(i):
            return jnp.where(own_is0, halves[0][i], halves[1][i])

        clsf = sel(0)
        cx = sel(1)
        cy = sel(2)
        tw = sel(3)
        th = sel(4)
        cell = sel(7)
        valid = (16 * half + lane) < T
        gx = (cx * wf).astype(jnp.int32)
        gy = (cy * hf).astype(jnp.int32)

        base = jnp.where(valid, cell, batch * HW)
        for s in range(C):
            tgidx[pl.ds(16 * s, 16)] = base + c16(s * CELLS)
        tgidx[pl.ds(16 * C, 16)] = base + (c16(5) + clsf.astype(jnp.int32)) * CELLS
        for e in range(NENT, NTCH * 128, 16):
            tgidx[pl.ds(e, 16)] = c16(4)
        pltpu.async_copy(flat_hbm.at[tgidx], tgbuf, sem_r)

        pltpu.make_async_copy(flat_hbm.at[tgidx], tgbuf, sem_r).wait()

        def slot(s):
            return tgbuf[pl.ds(16 * s, 16)]

        sx = 1.0 / (1.0 + jnp.exp(-slot(0)))
        sy = 1.0 / (1.0 + jnp.exp(-slot(1)))
        dx = sx - (cx * wf - gx.astype(jnp.float32))
        dy = sy - (cy * hf - gy.astype(jnp.float32))
        acc_xy = jnp.where(valid, (dx * dx + dy * dy) * 0.5, 0.0)
        dw = jnp.exp(slot(2)) - tw * wf
        dh = jnp.exp(slot(3)) - th * hf
        acc_wh = jnp.where(valid, (dw * dw + dh * dh) * 0.5, 0.0)
        csum = lax.fori_loop(
            0, NCLS,
            lambda j, a: a + _sp100(tgbuf[pl.ds(16 * j + 80, 16)]),
            zero16)
        kp, kn = _sp_both(slot(C))
        acc_cls = jnp.where(valid, csum + kn - kp, 0.0)
        cp_, cn_ = _sp_both(slot(4))
        c0m = jnp.where(halves[0][8], halves[0][7], -1 - lane)
        c1m = jnp.where(halves[1][8], halves[1][7], -33 - lane)
        own_m = jnp.where(own_is0, c0m, c1m)
        oth_m = jnp.where(own_is0, c1m, c0m)
        cnt = jnp.full((16,), 1.0, jnp.float32)
        one = jnp.full((16,), 1.0, jnp.float32)
        for s in range(16):
            if s > 0:
                cnt = cnt + jnp.where(own_m == _rot(own_m, lane, s), one, 0.0)
            cnt = cnt + jnp.where(own_m == _rot(oth_m, lane, s), one, 0.0)
        corr = jnp.where(valid, (cn_ - cp_) / cnt, 0.0)

        rv = jnp.where(lane == 1, _lanesum(acc_xy), zero16)
        rv = jnp.where(lane == 2, _lanesum(acc_wh), rv)
        rv = jnp.where(lane == 3, _lanesum(acc_cls), rv)
        rv = jnp.where(lane == 4, _lanesum(corr), rv)
        res_v[...] = rv

        pltpu.sync_copy(res_v, out_hbm.at[wid])

    return sc_fn


def kernel(predictions, targets, grid_size):
    B, HW, C = predictions.shape
    T = targets.shape[1]
    NCLS = C - 5
    TP = 32
    pt = jnp.transpose(predictions, (2, 0, 1))
    preds_flat = pt.reshape(C * B * HW)
    conf_dense = _conf_dense_tc(pt)
    tgt_t = jnp.transpose(targets, (0, 2, 1))
    tgt_p = jnp.concatenate(
        [tgt_t, jnp.zeros((B, 5, TP - T), tgt_t.dtype)], axis=-1
    ).reshape(B, 5 * TP)
    wf = grid_size[1].astype(jnp.float32)
    hf = grid_size[0].astype(jnp.float32)
    gridv = jnp.concatenate([jnp.full((16,), wf), jnp.full((16,), hf)])

    sc_fn = _build_sc_call(B, HW, C, T)
    out = sc_fn(preds_flat, tgt_p, gridv)

    sums = jnp.sum(out, axis=0)
    n_tgt = B * T
    loss_xy = sums[1] / n_tgt
    loss_wh = sums[2] / n_tgt
    loss_cls = sums[3] / (NCLS * n_tgt)
    loss_conf = (conf_dense + sums[4]) / (B * HW)
    total = loss_xy * 5.0 + loss_wh * 5.0 + loss_conf + loss_cls
    return (total, loss_xy, loss_wh, loss_conf, loss_cls)


# speedup vs baseline: 1.0065x; 1.0065x over previous
import functools

import jax
import jax.numpy as jnp
from jax import lax
from jax.experimental import pallas as pl
from jax.experimental.pallas import tpu as pltpu
from jax.experimental.pallas import tpu_sc as plsc

_LOG1P_C = (
    3.7050701212137938e-03,
    -2.2747693583369255e-02,
    6.5802522003650665e-02,
    -1.2435103952884674e-01,
    1.8400530517101288e-01,
    -2.4605530500411987e-01,
    3.3274200558662415e-01,
    -4.9995198845863342e-01,
    9.9999833106994629e-01,
    1.4770298761845880e-08,
)


def _log1p01(u):
    p = jnp.full((16,), _LOG1P_C[0], jnp.float32)
    for c in _LOG1P_C[1:]:
        p = p * u + c
    return p


def _sp100(x):
    l = _log1p01(jnp.exp(-jnp.abs(x)))
    return jnp.minimum(jnp.maximum(x, 0.0) + l, 100.0)


def _sp_both(x):
    l = _log1p01(jnp.exp(-jnp.abs(x)))
    sp_p = jnp.minimum(jnp.maximum(x, 0.0) + l, 100.0)
    sp_n = jnp.minimum(jnp.maximum(-x, 0.0) + l, 100.0)
    return sp_p, sp_n


def _rot(v, lane, s):
    return v.at[(lane + s) & 15].get(mode="promise_in_bounds")


def _lanesum(v):
    lane = lax.iota(jnp.int32, 16)
    for s in (1, 2, 4, 8):
        v = v + v.at[lane ^ s].get(mode="promise_in_bounds")
    return v


def _conf_dense_tc(pt):
    C_, B_, HW_ = pt.shape

    def body(ref, o_ref):
        x = ref[0]
        l = jnp.log1p(jnp.exp(-jnp.abs(x)))
        sp = jnp.minimum(jnp.maximum(x, 0.0) + l, 100.0)
        o_ref[0, 0] = jnp.sum(sp)

    return pl.pallas_call(
        body,
        grid=(1,),
        in_specs=[pl.BlockSpec((1, B_, HW_), lambda i: (4, 0, 0))],
        out_specs=pl.BlockSpec((1, 1), lambda i: (0, 0), memory_space=pltpu.SMEM),
        out_shape=jax.ShapeDtypeStruct((1, 1), jnp.float32),
    )(pt)[0, 0]


def _build_sc_call(B, HW, C, T):
    NCLS = C - 5
    NSLOT = C + 1
    NENT = NSLOT * 16
    NTCH = -(-NENT // 128)
    info = plsc.get_sparse_core_info()
    NC, NS = info.num_cores, info.num_subcores
    NW = NC * NS
    CELLS = B * HW
    TP = 32
    mesh = plsc.VectorSubcoreMesh(core_axis_name="c", subcore_axis_name="s")

    @functools.partial(
        pl.kernel,
        mesh=mesh,
        out_type=jax.ShapeDtypeStruct((NW, 16), jnp.float32),
        scratch_types=[
            pltpu.VMEM((NTCH * 128,), jnp.int32),
            pltpu.VMEM((NTCH * 128,), jnp.float32),
            pltpu.VMEM((5 * TP,), jnp.float32),
            pltpu.VMEM((32,), jnp.float32),
            pltpu.VMEM((16,), jnp.float32),
            pltpu.SemaphoreType.DMA,
        ],
    )
    def sc_fn(flat_hbm, tgt_hbm, grid_hbm, out_hbm,
              tgidx, tgbuf, tgt_v, grid_v, res_v, sem_r):
        wid = lax.axis_index("s") * NC + lax.axis_index("c")
        batch = wid // 2
        half = wid % 2
        lane = lax.iota(jnp.int32, 16)
        zero16 = jnp.zeros((16,), jnp.float32)
        c16 = lambda k: jnp.full((16,), k, jnp.int32)

        pltpu.sync_copy(tgt_hbm.at[batch], tgt_v)
        pltpu.sync_copy(grid_hbm, grid_v)
        wf = grid_v[pl.ds(0, 16)]
        hf = grid_v[pl.ds(16, 16)]
        wi = wf.astype(jnp.int32)

        halves = []
        for h2 in range(2):
            clsf = tgt_v[pl.ds(0 * TP + 16 * h2, 16)]
            cx = tgt_v[pl.ds(1 * TP + 16 * h2, 16)]
            cy = tgt_v[pl.ds(2 * TP + 16 * h2, 16)]
            tw = tgt_v[pl.ds(3 * TP + 16 * h2, 16)]
            th = tgt_v[pl.ds(4 * TP + 16 * h2, 16)]
            gx = (cx * wf).astype(jnp.int32)
            gy = (cy * hf).astype(jnp.int32)
            cell = gy * wi + gx + batch * HW
            valid = (lane + 16 * h2) < T
            halves.append((clsf, cx, cy, tw, th, gx, gy, cell, valid))

        own_is0 = half == 0

        def sel(i):
            return jnp.where(own_is0, halves[0][i], halves[1][i])

        clsf = sel(0)
        cx = sel(1)
        cy = sel(2)
        tw = sel(3)
        th = sel(4)
        cell = sel(7)
        valid = (16 * half + lane) < T
        gx = (cx * wf).astype(jnp.int32)
        gy = (cy * hf).astype(jnp.int32)

        base = jnp.where(valid, cell, batch * HW)
        for s in range(C):
            tgidx[pl.ds(16 * s, 16)] = base + c16(s * CELLS)
        tgidx[pl.ds(16 * C, 16)] = base + (c16(5) + clsf.astype(jnp.int32)) * CELLS
        for e in range(NENT, NTCH * 128, 16):
            tgidx[pl.ds(e, 16)] = c16(4)
        pltpu.async_copy(flat_hbm.at[tgidx], tgbuf, sem_r)

        pltpu.make_async_copy(flat_hbm.at[tgidx], tgbuf, sem_r).wait()

        def slot(s):
            return tgbuf[pl.ds(16 * s, 16)]

        sx = 1.0 / (1.0 + jnp.exp(-slot(0)))
        sy = 1.0 / (1.0 + jnp.exp(-slot(1)))
        dx = sx - (cx * wf - gx.astype(jnp.float32))
        dy = sy - (cy * hf - gy.astype(jnp.float32))
        acc_xy = jnp.where(valid, (dx * dx + dy * dy) * 0.5, 0.0)
        dw = jnp.exp(slot(2)) - tw * wf
        dh = jnp.exp(slot(3)) - th * hf
        acc_wh = jnp.where(valid, (dw * dw + dh * dh) * 0.5, 0.0)
        csum = lax.fori_loop(
            0, NCLS,
            lambda j, a: a + _sp100(tgbuf[pl.ds(16 * j + 80, 16)]),
            zero16)
        kp, kn = _sp_both(slot(C))
        acc_cls = jnp.where(valid, csum + kn - kp, 0.0)
        cp_, cn_ = _sp_both(slot(4))
        c0m = jnp.where(halves[0][8], halves[0][7], -1 - lane)
        c1m = jnp.where(halves[1][8], halves[1][7], -33 - lane)
        own_m = jnp.where(own_is0, c0m, c1m)
        oth_m = jnp.where(own_is0, c1m, c0m)
        cnt = jnp.full((16,), 1.0, jnp.float32)
        one = jnp.full((16,), 1.0, jnp.float32)
        for s in range(16):
            if s > 0:
                cnt = cnt + jnp.where(own_m == _rot(own_m, lane, s), one, 0.0)
            cnt = cnt + jnp.where(own_m == _rot(oth_m, lane, s), one, 0.0)
        corr = jnp.where(valid, (cn_ - cp_) / cnt, 0.0)

        rv = jnp.where(lane == 1, _lanesum(acc_xy), zero16)
        rv = jnp.where(lane == 2, _lanesum(acc_wh), rv)
        rv = jnp.where(lane == 3, _lanesum(acc_cls), rv)
        rv = jnp.where(lane == 4, _lanesum(corr), rv)
        res_v[...] = rv

        pltpu.sync_copy(res_v, out_hbm.at[wid])

    return sc_fn


def kernel(predictions, targets, grid_size):
    B, HW, C = predictions.shape
    T = targets.shape[1]
    NCLS = C - 5
    TP = 32
    pt = jnp.transpose(predictions, (2, 0, 1))
    preds_flat = pt.reshape(C * B * HW)
    conf_dense = _conf_dense_tc(pt)
    tgt_t = jnp.transpose(targets, (0, 2, 1))
    tgt_p = jnp.concatenate(
        [tgt_t, jnp.zeros((B, 5, TP - T), tgt_t.dtype)], axis=-1
    ).reshape(B, 5 * TP)
    wf = grid_size[1].astype(jnp.float32)
    hf = grid_size[0].astype(jnp.float32)
    gridv = jnp.concatenate([jnp.full((16,), wf), jnp.full((16,), hf)])

    sc_fn = _build_sc_call(B, HW, C, T)
    out = sc_fn(preds_flat, tgt_p, gridv)

    sums = jnp.sum(out, axis=0)
    n_tgt = B * T
    loss_xy = sums[1] / n_tgt
    loss_wh = sums[2] / n_tgt
    loss_cls = sums[3] / (NCLS * n_tgt)
    loss_conf = (conf_dense + sums[4]) / (B * HW)
    total = loss_xy * 5.0 + loss_wh * 5.0 + loss_conf + loss_cls
    return (total, loss_xy, loss_wh, loss_conf, loss_cls)
